# baseline (speedup 1.0000x reference)
import functools

import jax
import jax.numpy as jnp
from jax.experimental import pallas as pl
from jax.experimental.pallas import tpu as pltpu

_EPS = 1e-07


def _isd_kernel(conf_a, conf_b, mix_a, mix_b, loss_ref, acc_ref, *, pb, p_total, n_pblocks, n_pairs):
    i = pl.program_id(0)
    j = pl.program_id(1)

    @pl.when(jnp.logical_and(i == 0, j == 0))
    def _init():
        acc_ref[0] = 0.0
        acc_ref[1] = 0.0

    row = jax.lax.broadcasted_iota(jnp.int32, (pb, 1), 0)
    valid = (row + j * pb) < p_total

    def masked_kl(a_ref, b_ref, q_ref):
        a = a_ref[0]
        b = b_ref[0]
        mask_a = jnp.max(a[:, 1:], axis=1, keepdims=True) > a[:, 0:1]
        mask_b = jnp.max(b[:, 1:], axis=1, keepdims=True) > b[:, 0:1]
        only_a = jnp.logical_and(jnp.logical_and(mask_a, jnp.logical_not(mask_b)), valid)
        wa = only_a.astype(jnp.float32)
        t = jnp.where(valid, a, 1.0) + _EPS
        q = jnp.where(valid, q_ref[0], 1.0) + _EPS
        kl = t * (jnp.log(t) - jnp.log(q))
        return jnp.sum(kl * wa), jnp.sum(wa)

    s1, c1 = masked_kl(conf_a, conf_b, mix_a)
    s2, c2 = masked_kl(conf_b, conf_a, mix_b)
    acc_ref[0] += s1 + s2
    acc_ref[1] += c1 + c2

    @pl.when(jnp.logical_and(i == n_pairs - 1, j == n_pblocks - 1))
    def _finalize():
        total = acc_ref[0]
        cnt = acc_ref[1]
        val = jnp.where(cnt > 0.0, total / jnp.maximum(cnt, 1.0), 0.0)
        loss_ref[...] = jnp.full((1, 1), val, dtype=jnp.float32)


def kernel(args, lam, conf, loc, conf_mix, loc_mix):
    B, P, C = conf.shape
    half = B // 2
    PB = 2184
    n_pblocks = pl.cdiv(P, PB)

    blk = (1, PB, C)
    spec_a = pl.BlockSpec(blk, lambda i, j: (i, j, 0))
    spec_b = pl.BlockSpec(blk, lambda i, j: (i + half, j, 0))

    loss = pl.pallas_call(
        functools.partial(_isd_kernel, pb=PB, p_total=P, n_pblocks=n_pblocks, n_pairs=half),
        grid=(half, n_pblocks),
        in_specs=[spec_a, spec_b, spec_a, spec_b],
        out_specs=pl.BlockSpec((1, 1), lambda i, j: (0, 0)),
        out_shape=jax.ShapeDtypeStruct((1, 1), jnp.float32),
        scratch_shapes=[pltpu.SMEM((2,), jnp.float32)],
    )(conf, conf, conf_mix, conf_mix)

    return (jnp.zeros((1,), dtype=jnp.float32), loss[0, 0])
